# baseline (speedup 1.0000x reference)
import functools

import jax
import jax.numpy as jnp
from jax import lax
from jax.experimental import pallas as pl
from jax.experimental.pallas import tpu as pltpu
from jax.experimental.pallas import tpu_sc as plsc

N = 10000
NP = 10240
E = 320000
D = 128
H = 128

CHUNK = 96
NCHUNK = 3360
EP = NCHUNK * CHUNK
NSUB = 16
NBUF = 3
CH_PER_SUB = NCHUNK // NSUB
TRI = CH_PER_SUB // NBUF
ROWS_PER_SUB = NP // NSUB
ZROWS = 32
BN = 1280
BND = 1000

_mesh = lambda: plsc.VectorSubcoreMesh(core_axis_name="c", subcore_axis_name="s")


def _sc_degree(ei_chunks):
  @functools.partial(
      pl.kernel,
      out_type=jax.ShapeDtypeStruct((2, NP), jnp.float32),
      mesh=_mesh(),
      scratch_types=[
          pltpu.VMEM((2, CHUNK), jnp.int32),
          pltpu.VMEM((2, CHUNK), jnp.int32),
          pltpu.VMEM((2, CHUNK), jnp.int32),
          pltpu.VMEM((CHUNK,), jnp.float32),
          pltpu.VMEM((ROWS_PER_SUB,), jnp.float32),
          pltpu.VMEM_SHARED((NP,), jnp.float32),
          pltpu.SemaphoreType.DMA,
          pltpu.SemaphoreType.DMA,
          pltpu.SemaphoreType.DMA,
      ],
  )
  def deg_kernel(ei_hbm, deg_hbm, idx0_v, idx1_v, idx2_v, ones_v,
                 zbuf_v, hist_sh, ss0, ss1, ss2):
    c = lax.axis_index("c")
    s = lax.axis_index("s")

    def fill_ones(i, _):
      ones_v[pl.ds(i * 16, 16)] = jnp.full((16,), 1.0, jnp.float32)
      return 0
    lax.fori_loop(0, CHUNK // 16, fill_ones, 0)

    def fill_z(i, _):
      zbuf_v[pl.ds(i * 16, 16)] = jnp.zeros((16,), jnp.float32)
      return 0
    lax.fori_loop(0, ROWS_PER_SUB // 16, fill_z, 0)

    pltpu.sync_copy(zbuf_v, hist_sh.at[pl.ds(s * ROWS_PER_SUB, ROWS_PER_SUB)])
    plsc.subcore_barrier()

    base = s * CH_PER_SUB
    idxs = [idx0_v, idx1_v, idx2_v]
    sss = [ss0, ss1, ss2]
    for j in range(NBUF):
      pltpu.sync_copy(ei_hbm.at[base + j], idxs[j])
      pltpu.async_copy(ones_v, hist_sh.at[idxs[j].at[c]], sss[j], add=True)

    def tri(q, _):
      for j in range(NBUF):
        pltpu.make_async_copy(ones_v, hist_sh.at[idxs[j].at[c]], sss[j]).wait()

        @pl.when(q + 1 < TRI)
        def _():
          pltpu.sync_copy(ei_hbm.at[base + NBUF * (q + 1) + j], idxs[j])
          pltpu.async_copy(ones_v, hist_sh.at[idxs[j].at[c]], sss[j], add=True)
      return 0

    lax.fori_loop(0, TRI, tri, 0)
    plsc.subcore_barrier()

    sl = pl.ds(s * ROWS_PER_SUB, ROWS_PER_SUB)
    pltpu.sync_copy(hist_sh.at[sl], zbuf_v)
    pltpu.sync_copy(zbuf_v, deg_hbm.at[c, sl])

  return deg_kernel(ei_chunks)


def _sc_prop(h0, h1, ei_chunks):
  @functools.partial(
      pl.kernel,
      out_type=[jax.ShapeDtypeStruct((NP, H), jnp.float32),
                jax.ShapeDtypeStruct((NP, H), jnp.float32)],
      mesh=_mesh(),
      scratch_types=[
          pltpu.VMEM((2, CHUNK), jnp.int32),
          pltpu.VMEM((2, CHUNK), jnp.int32),
          pltpu.VMEM((2, CHUNK), jnp.int32),
          pltpu.VMEM((CHUNK, H), jnp.float32),
          pltpu.VMEM((CHUNK, H), jnp.float32),
          pltpu.VMEM((CHUNK, H), jnp.float32),
          pltpu.VMEM((ZROWS, H), jnp.float32),
          pltpu.VMEM_SHARED((NP, H), jnp.float32),
          pltpu.SemaphoreType.DMA,
          pltpu.SemaphoreType.DMA,
          pltpu.SemaphoreType.DMA,
          pltpu.SemaphoreType.DMA,
          pltpu.SemaphoreType.DMA,
          pltpu.SemaphoreType.DMA,
      ],
  )
  def prop_kernel(h0_hbm, h1_hbm, ei_hbm, out0_hbm, out1_hbm,
                  idx0_v, idx1_v, idx2_v,
                  rows0_v, rows1_v, rows2_v, zbuf_v, acc_sh,
                  gs0, gs1, gs2, ss0, ss1, ss2):
    c = lax.axis_index("c")
    s = lax.axis_index("s")

    def fill_z(t, _):
      zbuf_v[t // 8, pl.ds((t % 8) * 16, 16)] = jnp.zeros((16,), jnp.float32)
      return 0
    lax.fori_loop(0, ZROWS * (H // 16), fill_z, 0)

    def zero_acc(j, _):
      pltpu.sync_copy(zbuf_v,
                      acc_sh.at[pl.ds(s * ROWS_PER_SUB + j * ZROWS, ZROWS)])
      return 0
    lax.fori_loop(0, ROWS_PER_SUB // ZROWS, zero_acc, 0)
    plsc.subcore_barrier()

    base = s * CH_PER_SUB
    idxs = [idx0_v, idx1_v, idx2_v]
    rows = [rows0_v, rows1_v, rows2_v]
    gss = [gs0, gs1, gs2]
    sss = [ss0, ss1, ss2]

    def do_edges(h_hbm):
      for j in range(NBUF):
        pltpu.sync_copy(ei_hbm.at[base + j], idxs[j])
        pltpu.async_copy(h_hbm.at[idxs[j].at[0]], rows[j], gss[j])

      def tri(q, _):
        for j in range(NBUF):
          pltpu.make_async_copy(h_hbm.at[idxs[j].at[0]], rows[j],
                                gss[j]).wait()
          pltpu.async_copy(rows[j], acc_sh.at[idxs[j].at[1]], sss[j],
                           add=True)

        for j in range(NBUF):
          pltpu.make_async_copy(rows[j], acc_sh.at[idxs[j].at[1]],
                                sss[j]).wait()

          @pl.when(q + 1 < TRI)
          def _():
            pltpu.sync_copy(ei_hbm.at[base + NBUF * (q + 1) + j], idxs[j])
            pltpu.async_copy(h_hbm.at[idxs[j].at[0]], rows[j], gss[j])
        return 0

      lax.fori_loop(0, TRI, tri, 0)

    @pl.when(c == 0)
    def _():
      do_edges(h0_hbm)

    @pl.when(c == 1)
    def _():
      do_edges(h1_hbm)

    plsc.subcore_barrier()

    def copy_out(out_hbm):
      def co(j, _):
        sl = pl.ds(s * ROWS_PER_SUB + j * ZROWS, ZROWS)
        pltpu.sync_copy(acc_sh.at[sl], zbuf_v)
        pltpu.sync_copy(zbuf_v, out_hbm.at[sl])
        return 0
      lax.fori_loop(0, ROWS_PER_SUB // ZROWS, co, 0)

    @pl.when(c == 0)
    def _():
      copy_out(out0_hbm)

    @pl.when(c == 1)
    def _():
      copy_out(out1_hbm)

  return prop_kernel(h0, h1, ei_chunks)


def _norms(deg_ref, i):
  dout = deg_ref[0, pl.ds(i * BN, BN)]
  din = deg_ref[1, pl.ds(i * BN, BN)]
  ns = jnp.where(dout > 0, lax.rsqrt(dout), 0.0)[:, None]
  nd = jnp.where(din > 0, lax.rsqrt(din), 0.0)[:, None]
  return ns, nd


def _tc_mm_raw(x0, x1, W0, W1):
  def body(x0_ref, x1_ref, w0_ref, w1_ref, h0_ref, h1_ref):
    h0_ref[...] = jnp.dot(x0_ref[...], w0_ref[...],
                          preferred_element_type=jnp.float32)
    h1_ref[...] = jnp.dot(x1_ref[...], w1_ref[...],
                          preferred_element_type=jnp.float32)

  return pl.pallas_call(
      body,
      grid=(N // BND,),
      in_specs=[
          pl.BlockSpec((BND, D), lambda i: (i, 0)),
          pl.BlockSpec((BND, D), lambda i: (i, 0)),
          pl.BlockSpec((D, H), lambda i: (0, 0)),
          pl.BlockSpec((D, H), lambda i: (0, 0)),
      ],
      out_specs=[pl.BlockSpec((BND, H), lambda i: (i, 0)),
                 pl.BlockSpec((BND, H), lambda i: (i, 0))],
      out_shape=[jax.ShapeDtypeStruct((NP, H), jnp.float32),
                 jax.ShapeDtypeStruct((NP, H), jnp.float32)],
  )(x0, x1, W0, W1)


def _tc_scale(p0, p1, deg):
  def body(p0_ref, p1_ref, deg_ref, h0_ref, h1_ref):
    i = pl.program_id(0)
    ns, _ = _norms(deg_ref, i)
    h0_ref[...] = p0_ref[...] * ns
    h1_ref[...] = p1_ref[...] * ns

  return pl.pallas_call(
      body,
      grid=(NP // BN,),
      in_specs=[
          pl.BlockSpec((BN, H), lambda i: (i, 0)),
          pl.BlockSpec((BN, H), lambda i: (i, 0)),
          pl.BlockSpec((2, NP), lambda i: (0, 0)),
      ],
      out_specs=[pl.BlockSpec((BN, H), lambda i: (i, 0)),
                 pl.BlockSpec((BN, H), lambda i: (i, 0))],
      out_shape=[jax.ShapeDtypeStruct((NP, H), jnp.float32),
                 jax.ShapeDtypeStruct((NP, H), jnp.float32)],
  )(p0, p1, deg)


def _tc_mid(a0, a1, W0, W1, b1s, deg):
  def body(a0_ref, a1_ref, w0_ref, w1_ref, b_ref, deg_ref, g0_ref, g1_ref):
    i = pl.program_id(0)
    ns, nd = _norms(deg_ref, i)
    h0 = jax.nn.relu(a0_ref[...] * nd + b_ref[0][None, :])
    h1 = jax.nn.relu(a1_ref[...] * nd + b_ref[1][None, :])
    g0_ref[...] = jnp.dot(h0, w0_ref[...],
                          preferred_element_type=jnp.float32) * ns
    g1_ref[...] = jnp.dot(h1, w1_ref[...],
                          preferred_element_type=jnp.float32) * ns

  return pl.pallas_call(
      body,
      grid=(NP // BN,),
      in_specs=[
          pl.BlockSpec((BN, H), lambda i: (i, 0)),
          pl.BlockSpec((BN, H), lambda i: (i, 0)),
          pl.BlockSpec((H, H), lambda i: (0, 0)),
          pl.BlockSpec((H, H), lambda i: (0, 0)),
          pl.BlockSpec((2, H), lambda i: (0, 0)),
          pl.BlockSpec((2, NP), lambda i: (0, 0)),
      ],
      out_specs=[pl.BlockSpec((BN, H), lambda i: (i, 0)),
                 pl.BlockSpec((BN, H), lambda i: (i, 0))],
      out_shape=[jax.ShapeDtypeStruct((NP, H), jnp.float32),
                 jax.ShapeDtypeStruct((NP, H), jnp.float32)],
  )(a0, a1, W0, W1, b1s, deg)


def _elu(x):
  return jnp.where(x > 0, x, jnp.exp(x) - 1.0)


def _tc_znorm(a0, a1, deg, b2s):
  def body(a0_ref, a1_ref, deg_ref, b2_ref, z0_ref, z1_ref):
    i = pl.program_id(0)
    _, nd = _norms(deg_ref, i)
    z0_ref[...] = a0_ref[...] * nd + b2_ref[0][None, :]
    z1_ref[...] = a1_ref[...] * nd + b2_ref[1][None, :]

  return pl.pallas_call(
      body,
      grid=(NP // BN,),
      in_specs=[
          pl.BlockSpec((BN, H), lambda i: (i, 0)),
          pl.BlockSpec((BN, H), lambda i: (i, 0)),
          pl.BlockSpec((2, NP), lambda i: (0, 0)),
          pl.BlockSpec((2, H), lambda i: (0, 0)),
      ],
      out_specs=[pl.BlockSpec((BN, H), lambda i: (i, 0)),
                 pl.BlockSpec((BN, H), lambda i: (i, 0))],
      out_shape=[jax.ShapeDtypeStruct((NP, H), jnp.float32),
                 jax.ShapeDtypeStruct((NP, H), jnp.float32)],
  )(a0, a1, deg, b2s)


def _tc_decode(z0, z1, dW1, db1, dW2, db2):
  def body(z0_ref, z1_ref, w1_ref, bb1_ref, w2_ref, bb2_ref, out_ref):
    z0 = z0_ref[...]
    z1 = z1_ref[...]
    h0 = _elu(jnp.dot(z0, w1_ref[0], preferred_element_type=jnp.float32)
              + bb1_ref[0][None, :])
    d0 = _elu(jnp.dot(h0, w2_ref[0], preferred_element_type=jnp.float32)
              + bb2_ref[0][None, :])
    h1 = _elu(jnp.dot(z1, w1_ref[1], preferred_element_type=jnp.float32)
              + bb1_ref[1][None, :])
    d1 = _elu(jnp.dot(h1, w2_ref[1], preferred_element_type=jnp.float32)
              + bb2_ref[1][None, :])
    out_ref[0] = z0
    out_ref[1] = z1
    out_ref[2] = d0
    out_ref[3] = d1

  return pl.pallas_call(
      body,
      grid=(N // BND,),
      in_specs=[
          pl.BlockSpec((BND, H), lambda i: (i, 0)),
          pl.BlockSpec((BND, H), lambda i: (i, 0)),
          pl.BlockSpec((2, H, H), lambda i: (0, 0, 0)),
          pl.BlockSpec((2, H), lambda i: (0, 0)),
          pl.BlockSpec((2, H, H), lambda i: (0, 0, 0)),
          pl.BlockSpec((2, H), lambda i: (0, 0)),
      ],
      out_specs=pl.BlockSpec((4, BND, H), lambda i: (0, i, 0)),
      out_shape=jax.ShapeDtypeStruct((4, N, H), jnp.float32),
  )(z0, z1, dW1, db1, dW2, db2)


def kernel(x_F, x_S, edge_index, W1_F, b1_F, W2_F, b2_F, W1_S, b1_S, W2_S,
           b2_S, dFS_W1, dFS_b1, dFS_W2, dFS_b2, dSF_W1, dSF_b1, dSF_W2,
           dSF_b2):
  ei = edge_index.astype(jnp.int32)
  pad = N + (jnp.arange(EP - E, dtype=jnp.int32) % (NP - N))
  ei_pad = jnp.concatenate([ei, jnp.stack([pad, pad])], axis=1)
  ei_chunks = ei_pad.reshape(2, NCHUNK, CHUNK).transpose(1, 0, 2)

  deg = _sc_degree(ei_chunks)
  p0, p1 = _tc_mm_raw(x_F, x_S, W1_F, W1_S)
  h0, h1 = _tc_scale(p0, p1, deg)
  a0, a1 = _sc_prop(h0, h1, ei_chunks)
  g0, g1 = _tc_mid(a0, a1, W2_F, W2_S, jnp.stack([b1_F, b1_S]), deg)
  a20, a21 = _sc_prop(g0, g1, ei_chunks)
  z0, z1 = _tc_znorm(a20, a21, deg, jnp.stack([b2_F, b2_S]))
  return _tc_decode(z0, z1,
                    jnp.stack([dFS_W1, dSF_W1]), jnp.stack([dFS_b1, dSF_b1]),
                    jnp.stack([dFS_W2, dSF_W2]), jnp.stack([dFS_b2, dSF_b2]))


# speedup vs baseline: 1.0586x; 1.0586x over previous
import functools

import jax
import jax.numpy as jnp
from jax import lax
from jax.experimental import pallas as pl
from jax.experimental.pallas import tpu as pltpu
from jax.experimental.pallas import tpu_sc as plsc

N = 10000
NP = 10240
E = 320000
D = 128
H = 128

CHUNK = 128
NCHUNK = 2560
EP = NCHUNK * CHUNK
NSUB = 16
CH_PER_SUB = NCHUNK // NSUB
PAIRS = CH_PER_SUB // 2
ROWS_PER_SUB = NP // NSUB
ZROWS = 64
BN = 1280
BND = 1000

_mesh = lambda: plsc.VectorSubcoreMesh(core_axis_name="c", subcore_axis_name="s")


def _sc_degree(ei_chunks):
  @functools.partial(
      pl.kernel,
      out_type=jax.ShapeDtypeStruct((2, NP), jnp.float32),
      mesh=_mesh(),
      scratch_types=[
          pltpu.VMEM((2, CHUNK), jnp.int32),
          pltpu.VMEM((2, CHUNK), jnp.int32),
          pltpu.VMEM((CHUNK,), jnp.float32),
          pltpu.VMEM((ROWS_PER_SUB,), jnp.float32),
          pltpu.VMEM_SHARED((NP,), jnp.float32),
          pltpu.SemaphoreType.DMA,
          pltpu.SemaphoreType.DMA,
      ],
  )
  def deg_kernel(ei_hbm, deg_hbm, idx0_v, idx1_v, ones_v, zbuf_v, hist_sh,
                 ss0, ss1):
    c = lax.axis_index("c")
    s = lax.axis_index("s")

    def fill_ones(i, _):
      ones_v[pl.ds(i * 16, 16)] = jnp.full((16,), 1.0, jnp.float32)
      return 0
    lax.fori_loop(0, CHUNK // 16, fill_ones, 0)

    def fill_z(i, _):
      zbuf_v[pl.ds(i * 16, 16)] = jnp.zeros((16,), jnp.float32)
      return 0
    lax.fori_loop(0, ROWS_PER_SUB // 16, fill_z, 0)

    pltpu.sync_copy(zbuf_v, hist_sh.at[pl.ds(s * ROWS_PER_SUB, ROWS_PER_SUB)])
    plsc.subcore_barrier()

    base = s * CH_PER_SUB
    pltpu.sync_copy(ei_hbm.at[base], idx0_v)

    def pair(p, _):
      pltpu.async_copy(ones_v, hist_sh.at[idx0_v.at[c]], ss0, add=True)

      @pl.when(p >= 1)
      def _():
        pltpu.make_async_copy(ones_v, hist_sh.at[idx1_v.at[c]], ss1).wait()
      pltpu.sync_copy(ei_hbm.at[base + 2 * p + 1], idx1_v)
      pltpu.async_copy(ones_v, hist_sh.at[idx1_v.at[c]], ss1, add=True)
      pltpu.make_async_copy(ones_v, hist_sh.at[idx0_v.at[c]], ss0).wait()

      @pl.when(p + 1 < PAIRS)
      def _():
        pltpu.sync_copy(ei_hbm.at[base + 2 * p + 2], idx0_v)
      return 0

    lax.fori_loop(0, PAIRS, pair, 0)
    pltpu.make_async_copy(ones_v, hist_sh.at[idx1_v.at[c]], ss1).wait()
    plsc.subcore_barrier()

    sl = pl.ds(s * ROWS_PER_SUB, ROWS_PER_SUB)
    pltpu.sync_copy(hist_sh.at[sl], zbuf_v)
    pltpu.sync_copy(zbuf_v, deg_hbm.at[c, sl])

  return deg_kernel(ei_chunks)


def _sc_prop(h0, h1, ei_chunks):
  @functools.partial(
      pl.kernel,
      out_type=[jax.ShapeDtypeStruct((NP, H), jnp.float32),
                jax.ShapeDtypeStruct((NP, H), jnp.float32)],
      mesh=_mesh(),
      scratch_types=[
          pltpu.VMEM((2, CHUNK), jnp.int32),
          pltpu.VMEM((2, CHUNK), jnp.int32),
          pltpu.VMEM((CHUNK, H), jnp.float32),
          pltpu.VMEM((CHUNK, H), jnp.float32),
          pltpu.VMEM((ZROWS, H), jnp.float32),
          pltpu.VMEM_SHARED((NP, H), jnp.float32),
          pltpu.SemaphoreType.DMA,
          pltpu.SemaphoreType.DMA,
          pltpu.SemaphoreType.DMA,
          pltpu.SemaphoreType.DMA,
      ],
  )
  def prop_kernel(h0_hbm, h1_hbm, ei_hbm, out0_hbm, out1_hbm,
                  idx0_v, idx1_v, rows0_v, rows1_v, zbuf_v, acc_sh,
                  gs0, gs1, ss0, ss1):
    c = lax.axis_index("c")
    s = lax.axis_index("s")

    def fill_z(t, _):
      zbuf_v[t // 8, pl.ds((t % 8) * 16, 16)] = jnp.zeros((16,), jnp.float32)
      return 0
    lax.fori_loop(0, ZROWS * (H // 16), fill_z, 0)

    def zero_acc(j, _):
      pltpu.sync_copy(zbuf_v,
                      acc_sh.at[pl.ds(s * ROWS_PER_SUB + j * ZROWS, ZROWS)])
      return 0
    lax.fori_loop(0, ROWS_PER_SUB // ZROWS, zero_acc, 0)
    plsc.subcore_barrier()

    base = s * CH_PER_SUB

    def do_edges(h_hbm):
      pltpu.sync_copy(ei_hbm.at[base], idx0_v)
      pltpu.async_copy(h_hbm.at[idx0_v.at[0]], rows0_v, gs0)

      def pair(p, _):
        @pl.when(p >= 1)
        def _():
          pltpu.make_async_copy(rows1_v, acc_sh.at[idx1_v.at[1]], ss1).wait()
        pltpu.sync_copy(ei_hbm.at[base + 2 * p + 1], idx1_v)
        pltpu.async_copy(h_hbm.at[idx1_v.at[0]], rows1_v, gs1)
        pltpu.make_async_copy(h_hbm.at[idx0_v.at[0]], rows0_v, gs0).wait()
        pltpu.async_copy(rows0_v, acc_sh.at[idx0_v.at[1]], ss0, add=True)

        pltpu.make_async_copy(rows0_v, acc_sh.at[idx0_v.at[1]], ss0).wait()

        @pl.when(p + 1 < PAIRS)
        def _():
          pltpu.sync_copy(ei_hbm.at[base + 2 * p + 2], idx0_v)
          pltpu.async_copy(h_hbm.at[idx0_v.at[0]], rows0_v, gs0)
        pltpu.make_async_copy(h_hbm.at[idx1_v.at[0]], rows1_v, gs1).wait()
        pltpu.async_copy(rows1_v, acc_sh.at[idx1_v.at[1]], ss1, add=True)
        return 0

      lax.fori_loop(0, PAIRS, pair, 0)
      pltpu.make_async_copy(rows1_v, acc_sh.at[idx1_v.at[1]], ss1).wait()

    @pl.when(c == 0)
    def _():
      do_edges(h0_hbm)

    @pl.when(c == 1)
    def _():
      do_edges(h1_hbm)

    plsc.subcore_barrier()

    def copy_out(out_hbm):
      def co(j, _):
        sl = pl.ds(s * ROWS_PER_SUB + j * ZROWS, ZROWS)
        pltpu.sync_copy(acc_sh.at[sl], zbuf_v)
        pltpu.sync_copy(zbuf_v, out_hbm.at[sl])
        return 0
      lax.fori_loop(0, ROWS_PER_SUB // ZROWS, co, 0)

    @pl.when(c == 0)
    def _():
      copy_out(out0_hbm)

    @pl.when(c == 1)
    def _():
      copy_out(out1_hbm)

  return prop_kernel(h0, h1, ei_chunks)


def _norms(deg_ref, i):
  dout = deg_ref[0, pl.ds(i * BN, BN)]
  din = deg_ref[1, pl.ds(i * BN, BN)]
  ns = jnp.where(dout > 0, lax.rsqrt(dout), 0.0)[:, None]
  nd = jnp.where(din > 0, lax.rsqrt(din), 0.0)[:, None]
  return ns, nd


def _tc_scale_x(x0, x1, deg):
  def body(x0_ref, x1_ref, deg_ref, h0_ref, h1_ref):
    i = pl.program_id(0)
    ns, _ = _norms(deg_ref, i)
    h0_ref[...] = x0_ref[...] * ns
    h1_ref[...] = x1_ref[...] * ns

  return pl.pallas_call(
      body,
      grid=(NP // BN,),
      in_specs=[
          pl.BlockSpec((BN, D), lambda i: (i, 0)),
          pl.BlockSpec((BN, D), lambda i: (i, 0)),
          pl.BlockSpec((2, NP), lambda i: (0, 0)),
      ],
      out_specs=[pl.BlockSpec((BN, D), lambda i: (i, 0)),
                 pl.BlockSpec((BN, D), lambda i: (i, 0))],
      out_shape=[jax.ShapeDtypeStruct((NP, D), jnp.float32),
                 jax.ShapeDtypeStruct((NP, D), jnp.float32)],
  )(x0, x1, deg)


def _tc_mid(a0, a1, W10, W11, b1s, deg):
  def body(a0_ref, a1_ref, w0_ref, w1_ref, b_ref, deg_ref, g0_ref, g1_ref):
    i = pl.program_id(0)
    ns, nd = _norms(deg_ref, i)
    h0 = jax.nn.relu(jnp.dot(a0_ref[...] * nd, w0_ref[...],
                             preferred_element_type=jnp.float32)
                     + b_ref[0][None, :])
    h1 = jax.nn.relu(jnp.dot(a1_ref[...] * nd, w1_ref[...],
                             preferred_element_type=jnp.float32)
                     + b_ref[1][None, :])
    g0_ref[...] = h0 * ns
    g1_ref[...] = h1 * ns

  return pl.pallas_call(
      body,
      grid=(NP // BN,),
      in_specs=[
          pl.BlockSpec((BN, D), lambda i: (i, 0)),
          pl.BlockSpec((BN, D), lambda i: (i, 0)),
          pl.BlockSpec((D, H), lambda i: (0, 0)),
          pl.BlockSpec((D, H), lambda i: (0, 0)),
          pl.BlockSpec((2, H), lambda i: (0, 0)),
          pl.BlockSpec((2, NP), lambda i: (0, 0)),
      ],
      out_specs=[pl.BlockSpec((BN, H), lambda i: (i, 0)),
                 pl.BlockSpec((BN, H), lambda i: (i, 0))],
      out_shape=[jax.ShapeDtypeStruct((NP, H), jnp.float32),
                 jax.ShapeDtypeStruct((NP, H), jnp.float32)],
  )(a0, a1, W10, W11, b1s, deg)


def _tc_znorm(a0, a1, W20, W21, b2s, deg):
  def body(a0_ref, a1_ref, w0_ref, w1_ref, b2_ref, deg_ref, z0_ref, z1_ref):
    i = pl.program_id(0)
    _, nd = _norms(deg_ref, i)
    z0_ref[...] = jnp.dot(a0_ref[...] * nd, w0_ref[...],
                          preferred_element_type=jnp.float32) + b2_ref[0][None, :]
    z1_ref[...] = jnp.dot(a1_ref[...] * nd, w1_ref[...],
                          preferred_element_type=jnp.float32) + b2_ref[1][None, :]

  return pl.pallas_call(
      body,
      grid=(NP // BN,),
      in_specs=[
          pl.BlockSpec((BN, H), lambda i: (i, 0)),
          pl.BlockSpec((BN, H), lambda i: (i, 0)),
          pl.BlockSpec((H, H), lambda i: (0, 0)),
          pl.BlockSpec((H, H), lambda i: (0, 0)),
          pl.BlockSpec((2, H), lambda i: (0, 0)),
          pl.BlockSpec((2, NP), lambda i: (0, 0)),
      ],
      out_specs=[pl.BlockSpec((BN, H), lambda i: (i, 0)),
                 pl.BlockSpec((BN, H), lambda i: (i, 0))],
      out_shape=[jax.ShapeDtypeStruct((NP, H), jnp.float32),
                 jax.ShapeDtypeStruct((NP, H), jnp.float32)],
  )(a0, a1, W20, W21, b2s, deg)


def _elu(x):
  return jnp.where(x > 0, x, jnp.exp(x) - 1.0)


def _tc_decode(z0, z1, dW1, db1, dW2, db2):
  def body(z0_ref, z1_ref, w1_ref, bb1_ref, w2_ref, bb2_ref, out_ref):
    z0 = z0_ref[...]
    z1 = z1_ref[...]
    h0 = _elu(jnp.dot(z0, w1_ref[0], preferred_element_type=jnp.float32)
              + bb1_ref[0][None, :])
    d0 = _elu(jnp.dot(h0, w2_ref[0], preferred_element_type=jnp.float32)
              + bb2_ref[0][None, :])
    h1 = _elu(jnp.dot(z1, w1_ref[1], preferred_element_type=jnp.float32)
              + bb1_ref[1][None, :])
    d1 = _elu(jnp.dot(h1, w2_ref[1], preferred_element_type=jnp.float32)
              + bb2_ref[1][None, :])
    out_ref[0] = z0
    out_ref[1] = z1
    out_ref[2] = d0
    out_ref[3] = d1

  return pl.pallas_call(
      body,
      grid=(N // BND,),
      in_specs=[
          pl.BlockSpec((BND, H), lambda i: (i, 0)),
          pl.BlockSpec((BND, H), lambda i: (i, 0)),
          pl.BlockSpec((2, H, H), lambda i: (0, 0, 0)),
          pl.BlockSpec((2, H), lambda i: (0, 0)),
          pl.BlockSpec((2, H, H), lambda i: (0, 0, 0)),
          pl.BlockSpec((2, H), lambda i: (0, 0)),
      ],
      out_specs=pl.BlockSpec((4, BND, H), lambda i: (0, i, 0)),
      out_shape=jax.ShapeDtypeStruct((4, N, H), jnp.float32),
  )(z0, z1, dW1, db1, dW2, db2)


def kernel(x_F, x_S, edge_index, W1_F, b1_F, W2_F, b2_F, W1_S, b1_S, W2_S,
           b2_S, dFS_W1, dFS_b1, dFS_W2, dFS_b2, dSF_W1, dSF_b1, dSF_W2,
           dSF_b2):
  ei = edge_index.astype(jnp.int32)
  pad = N + (jnp.arange(EP - E, dtype=jnp.int32) % (NP - N))
  ei_pad = jnp.concatenate([ei, jnp.stack([pad, pad])], axis=1)
  ei_chunks = ei_pad.reshape(2, NCHUNK, CHUNK).transpose(1, 0, 2)

  deg = _sc_degree(ei_chunks)
  sx0, sx1 = _tc_scale_x(x_F, x_S, deg)
  a0, a1 = _sc_prop(sx0, sx1, ei_chunks)
  g0, g1 = _tc_mid(a0, a1, W1_F, W1_S, jnp.stack([b1_F, b1_S]), deg)
  a20, a21 = _sc_prop(g0, g1, ei_chunks)
  z0, z1 = _tc_znorm(a20, a21, W2_F, W2_S, jnp.stack([b2_F, b2_S]), deg)
  return _tc_decode(z0, z1,
                    jnp.stack([dFS_W1, dSF_W1]), jnp.stack([dFS_b1, dSF_b1]),
                    jnp.stack([dFS_W2, dSF_W2]), jnp.stack([dFS_b2, dSF_b2]))


# speedup vs baseline: 1.0600x; 1.0013x over previous
import functools

import jax
import jax.numpy as jnp
from jax import lax
from jax.experimental import pallas as pl
from jax.experimental.pallas import tpu as pltpu
from jax.experimental.pallas import tpu_sc as plsc

N = 10000
NP = 10240
E = 320000
D = 128
H = 128

CHUNK = 128
NCHUNK = 2560
EP = NCHUNK * CHUNK
NSUB = 16
CH_PER_SUB = NCHUNK // NSUB
PAIRS = CH_PER_SUB // 2
ROWS_PER_SUB = NP // NSUB
ZROWS = 64
BN = 1280
BND = 1000

_mesh = lambda: plsc.VectorSubcoreMesh(core_axis_name="c", subcore_axis_name="s")


def _sc_degree(ei_chunks):
  @functools.partial(
      pl.kernel,
      out_type=jax.ShapeDtypeStruct((2, NP), jnp.float32),
      mesh=_mesh(),
      scratch_types=[
          pltpu.VMEM((2, CHUNK), jnp.int32),
          pltpu.VMEM((2, CHUNK), jnp.int32),
          pltpu.VMEM((CHUNK,), jnp.float32),
          pltpu.VMEM((ROWS_PER_SUB,), jnp.float32),
          pltpu.VMEM_SHARED((NP,), jnp.float32),
          pltpu.SemaphoreType.DMA,
          pltpu.SemaphoreType.DMA,
      ],
  )
  def deg_kernel(ei_hbm, deg_hbm, idx0_v, idx1_v, ones_v, zbuf_v, hist_sh,
                 ss0, ss1):
    c = lax.axis_index("c")
    s = lax.axis_index("s")

    def fill_ones(i, _):
      ones_v[pl.ds(i * 16, 16)] = jnp.full((16,), 1.0, jnp.float32)
      return 0
    lax.fori_loop(0, CHUNK // 16, fill_ones, 0)

    def fill_z(i, _):
      zbuf_v[pl.ds(i * 16, 16)] = jnp.zeros((16,), jnp.float32)
      return 0
    lax.fori_loop(0, ROWS_PER_SUB // 16, fill_z, 0)

    pltpu.sync_copy(zbuf_v, hist_sh.at[pl.ds(s * ROWS_PER_SUB, ROWS_PER_SUB)])
    plsc.subcore_barrier()

    base = s * CH_PER_SUB
    pltpu.sync_copy(ei_hbm.at[base], idx0_v)

    def pair(p, _):
      pltpu.async_copy(ones_v, hist_sh.at[idx0_v.at[c]], ss0, add=True)

      @pl.when(p >= 1)
      def _():
        pltpu.make_async_copy(ones_v, hist_sh.at[idx1_v.at[c]], ss1).wait()
      pltpu.sync_copy(ei_hbm.at[base + 2 * p + 1], idx1_v)
      pltpu.async_copy(ones_v, hist_sh.at[idx1_v.at[c]], ss1, add=True)
      pltpu.make_async_copy(ones_v, hist_sh.at[idx0_v.at[c]], ss0).wait()

      @pl.when(p + 1 < PAIRS)
      def _():
        pltpu.sync_copy(ei_hbm.at[base + 2 * p + 2], idx0_v)
      return 0

    lax.fori_loop(0, PAIRS, pair, 0)
    pltpu.make_async_copy(ones_v, hist_sh.at[idx1_v.at[c]], ss1).wait()
    plsc.subcore_barrier()

    sl = pl.ds(s * ROWS_PER_SUB, ROWS_PER_SUB)
    pltpu.sync_copy(hist_sh.at[sl], deg_hbm.at[c, sl])

  return deg_kernel(ei_chunks)


def _sc_prop(h0, h1, ei_chunks):
  @functools.partial(
      pl.kernel,
      out_type=[jax.ShapeDtypeStruct((NP, H), jnp.float32),
                jax.ShapeDtypeStruct((NP, H), jnp.float32)],
      mesh=_mesh(),
      scratch_types=[
          pltpu.VMEM((2, CHUNK), jnp.int32),
          pltpu.VMEM((2, CHUNK), jnp.int32),
          pltpu.VMEM((CHUNK, H), jnp.float32),
          pltpu.VMEM((CHUNK, H), jnp.float32),
          pltpu.VMEM((ZROWS, H), jnp.float32),
          pltpu.VMEM_SHARED((NP, H), jnp.float32),
          pltpu.SemaphoreType.DMA,
          pltpu.SemaphoreType.DMA,
          pltpu.SemaphoreType.DMA,
          pltpu.SemaphoreType.DMA,
      ],
  )
  def prop_kernel(h0_hbm, h1_hbm, ei_hbm, out0_hbm, out1_hbm,
                  idx0_v, idx1_v, rows0_v, rows1_v, zbuf_v, acc_sh,
                  gs0, gs1, ss0, ss1):
    c = lax.axis_index("c")
    s = lax.axis_index("s")

    def fill_z(t, _):
      zbuf_v[t // 8, pl.ds((t % 8) * 16, 16)] = jnp.zeros((16,), jnp.float32)
      return 0
    lax.fori_loop(0, ZROWS * (H // 16), fill_z, 0)

    def zero_acc(j, _):
      pltpu.sync_copy(zbuf_v,
                      acc_sh.at[pl.ds(s * ROWS_PER_SUB + j * ZROWS, ZROWS)])
      return 0
    lax.fori_loop(0, ROWS_PER_SUB // ZROWS, zero_acc, 0)
    plsc.subcore_barrier()

    base = s * CH_PER_SUB

    def do_edges(h_hbm):
      pltpu.sync_copy(ei_hbm.at[base], idx0_v)
      pltpu.async_copy(h_hbm.at[idx0_v.at[0]], rows0_v, gs0)

      def pair(p, _):
        @pl.when(p >= 1)
        def _():
          pltpu.make_async_copy(rows1_v, acc_sh.at[idx1_v.at[1]], ss1).wait()
        pltpu.sync_copy(ei_hbm.at[base + 2 * p + 1], idx1_v)
        pltpu.async_copy(h_hbm.at[idx1_v.at[0]], rows1_v, gs1)
        pltpu.make_async_copy(h_hbm.at[idx0_v.at[0]], rows0_v, gs0).wait()
        pltpu.async_copy(rows0_v, acc_sh.at[idx0_v.at[1]], ss0, add=True)

        pltpu.make_async_copy(rows0_v, acc_sh.at[idx0_v.at[1]], ss0).wait()

        @pl.when(p + 1 < PAIRS)
        def _():
          pltpu.sync_copy(ei_hbm.at[base + 2 * p + 2], idx0_v)
          pltpu.async_copy(h_hbm.at[idx0_v.at[0]], rows0_v, gs0)
        pltpu.make_async_copy(h_hbm.at[idx1_v.at[0]], rows1_v, gs1).wait()
        pltpu.async_copy(rows1_v, acc_sh.at[idx1_v.at[1]], ss1, add=True)
        return 0

      lax.fori_loop(0, PAIRS, pair, 0)
      pltpu.make_async_copy(rows1_v, acc_sh.at[idx1_v.at[1]], ss1).wait()

    @pl.when(c == 0)
    def _():
      do_edges(h0_hbm)

    @pl.when(c == 1)
    def _():
      do_edges(h1_hbm)

    plsc.subcore_barrier()

    def copy_out(out_hbm):
      sl = pl.ds(s * ROWS_PER_SUB, ROWS_PER_SUB)
      pltpu.sync_copy(acc_sh.at[sl], out_hbm.at[sl])

    @pl.when(c == 0)
    def _():
      copy_out(out0_hbm)

    @pl.when(c == 1)
    def _():
      copy_out(out1_hbm)

  return prop_kernel(h0, h1, ei_chunks)


def _norms(deg_ref, i):
  dout = deg_ref[0, pl.ds(i * BN, BN)]
  din = deg_ref[1, pl.ds(i * BN, BN)]
  ns = jnp.where(dout > 0, lax.rsqrt(dout), 0.0)[:, None]
  nd = jnp.where(din > 0, lax.rsqrt(din), 0.0)[:, None]
  return ns, nd


def _tc_scale_x(x0, x1, deg):
  def body(x0_ref, x1_ref, deg_ref, h0_ref, h1_ref):
    i = pl.program_id(0)
    ns, _ = _norms(deg_ref, i)
    h0_ref[...] = x0_ref[...] * ns
    h1_ref[...] = x1_ref[...] * ns

  return pl.pallas_call(
      body,
      grid=(NP // BN,),
      in_specs=[
          pl.BlockSpec((BN, D), lambda i: (i, 0)),
          pl.BlockSpec((BN, D), lambda i: (i, 0)),
          pl.BlockSpec((2, NP), lambda i: (0, 0)),
      ],
      out_specs=[pl.BlockSpec((BN, D), lambda i: (i, 0)),
                 pl.BlockSpec((BN, D), lambda i: (i, 0))],
      out_shape=[jax.ShapeDtypeStruct((NP, D), jnp.float32),
                 jax.ShapeDtypeStruct((NP, D), jnp.float32)],
  )(x0, x1, deg)


def _tc_mid(a0, a1, W10, W11, b1s, deg):
  def body(a0_ref, a1_ref, w0_ref, w1_ref, b_ref, deg_ref, g0_ref, g1_ref):
    i = pl.program_id(0)
    ns, nd = _norms(deg_ref, i)
    h0 = jax.nn.relu(jnp.dot(a0_ref[...] * nd, w0_ref[...],
                             preferred_element_type=jnp.float32)
                     + b_ref[0][None, :])
    h1 = jax.nn.relu(jnp.dot(a1_ref[...] * nd, w1_ref[...],
                             preferred_element_type=jnp.float32)
                     + b_ref[1][None, :])
    g0_ref[...] = h0 * ns
    g1_ref[...] = h1 * ns

  return pl.pallas_call(
      body,
      grid=(NP // BN,),
      in_specs=[
          pl.BlockSpec((BN, D), lambda i: (i, 0)),
          pl.BlockSpec((BN, D), lambda i: (i, 0)),
          pl.BlockSpec((D, H), lambda i: (0, 0)),
          pl.BlockSpec((D, H), lambda i: (0, 0)),
          pl.BlockSpec((2, H), lambda i: (0, 0)),
          pl.BlockSpec((2, NP), lambda i: (0, 0)),
      ],
      out_specs=[pl.BlockSpec((BN, H), lambda i: (i, 0)),
                 pl.BlockSpec((BN, H), lambda i: (i, 0))],
      out_shape=[jax.ShapeDtypeStruct((NP, H), jnp.float32),
                 jax.ShapeDtypeStruct((NP, H), jnp.float32)],
  )(a0, a1, W10, W11, b1s, deg)


def _tc_znorm(a0, a1, W20, W21, b2s, deg):
  def body(a0_ref, a1_ref, w0_ref, w1_ref, b2_ref, deg_ref, z0_ref, z1_ref):
    i = pl.program_id(0)
    _, nd = _norms(deg_ref, i)
    z0_ref[...] = jnp.dot(a0_ref[...] * nd, w0_ref[...],
                          preferred_element_type=jnp.float32) + b2_ref[0][None, :]
    z1_ref[...] = jnp.dot(a1_ref[...] * nd, w1_ref[...],
                          preferred_element_type=jnp.float32) + b2_ref[1][None, :]

  return pl.pallas_call(
      body,
      grid=(NP // BN,),
      in_specs=[
          pl.BlockSpec((BN, H), lambda i: (i, 0)),
          pl.BlockSpec((BN, H), lambda i: (i, 0)),
          pl.BlockSpec((H, H), lambda i: (0, 0)),
          pl.BlockSpec((H, H), lambda i: (0, 0)),
          pl.BlockSpec((2, H), lambda i: (0, 0)),
          pl.BlockSpec((2, NP), lambda i: (0, 0)),
      ],
      out_specs=[pl.BlockSpec((BN, H), lambda i: (i, 0)),
                 pl.BlockSpec((BN, H), lambda i: (i, 0))],
      out_shape=[jax.ShapeDtypeStruct((NP, H), jnp.float32),
                 jax.ShapeDtypeStruct((NP, H), jnp.float32)],
  )(a0, a1, W20, W21, b2s, deg)


def _elu(x):
  return jnp.where(x > 0, x, jnp.exp(x) - 1.0)


def _tc_decode(z0, z1, dW1, db1, dW2, db2):
  def body(z0_ref, z1_ref, w1_ref, bb1_ref, w2_ref, bb2_ref, out_ref):
    z0 = z0_ref[...]
    z1 = z1_ref[...]
    h0 = _elu(jnp.dot(z0, w1_ref[0], preferred_element_type=jnp.float32)
              + bb1_ref[0][None, :])
    d0 = _elu(jnp.dot(h0, w2_ref[0], preferred_element_type=jnp.float32)
              + bb2_ref[0][None, :])
    h1 = _elu(jnp.dot(z1, w1_ref[1], preferred_element_type=jnp.float32)
              + bb1_ref[1][None, :])
    d1 = _elu(jnp.dot(h1, w2_ref[1], preferred_element_type=jnp.float32)
              + bb2_ref[1][None, :])
    out_ref[0] = z0
    out_ref[1] = z1
    out_ref[2] = d0
    out_ref[3] = d1

  return pl.pallas_call(
      body,
      grid=(N // BND,),
      in_specs=[
          pl.BlockSpec((BND, H), lambda i: (i, 0)),
          pl.BlockSpec((BND, H), lambda i: (i, 0)),
          pl.BlockSpec((2, H, H), lambda i: (0, 0, 0)),
          pl.BlockSpec((2, H), lambda i: (0, 0)),
          pl.BlockSpec((2, H, H), lambda i: (0, 0, 0)),
          pl.BlockSpec((2, H), lambda i: (0, 0)),
      ],
      out_specs=pl.BlockSpec((4, BND, H), lambda i: (0, i, 0)),
      out_shape=jax.ShapeDtypeStruct((4, N, H), jnp.float32),
  )(z0, z1, dW1, db1, dW2, db2)


def kernel(x_F, x_S, edge_index, W1_F, b1_F, W2_F, b2_F, W1_S, b1_S, W2_S,
           b2_S, dFS_W1, dFS_b1, dFS_W2, dFS_b2, dSF_W1, dSF_b1, dSF_W2,
           dSF_b2):
  ei = edge_index.astype(jnp.int32)
  pad = N + (jnp.arange(EP - E, dtype=jnp.int32) % (NP - N))
  ei_pad = jnp.concatenate([ei, jnp.stack([pad, pad])], axis=1)
  ei_chunks = ei_pad.reshape(2, NCHUNK, CHUNK).transpose(1, 0, 2)

  deg = _sc_degree(ei_chunks)
  sx0, sx1 = _tc_scale_x(x_F, x_S, deg)
  a0, a1 = _sc_prop(sx0, sx1, ei_chunks)
  g0, g1 = _tc_mid(a0, a1, W1_F, W1_S, jnp.stack([b1_F, b1_S]), deg)
  a20, a21 = _sc_prop(g0, g1, ei_chunks)
  z0, z1 = _tc_znorm(a20, a21, W2_F, W2_S, jnp.stack([b2_F, b2_S]), deg)
  return _tc_decode(z0, z1,
                    jnp.stack([dFS_W1, dSF_W1]), jnp.stack([dFS_b1, dSF_b1]),
                    jnp.stack([dFS_W2, dSF_W2]), jnp.stack([dFS_b2, dSF_b2]))
